# Initial kernel scaffold; baseline (speedup 1.0000x reference)
import jax
import jax.numpy as jnp
from jax.experimental import pallas as pl


def kernel(x, edge_index, edge_attr, batch, params):
    raise NotImplementedError("write your pallas kernel here")


# speedup vs baseline: 2.2557x; 2.2557x over previous
import functools

import jax
import jax.numpy as jnp
from jax import lax
from jax.experimental import pallas as pl
from jax.experimental.pallas import tpu as pltpu
from jax.experimental.pallas import tpu_sc as plsc

F32 = jnp.float32
N_NODES = 10000
N_EDGES = 320000
HIDDEN = 128
EDGE_DIM = 16
N_GRAPHS = 16
N_BLOBS = 8
BN_EPS = 1e-5
LN_EPS = 1e-5

NC = 2
NS = 16
NW = NC * NS

E_PAD = 327680
PAD_N = E_PAD - N_EDGES
ROWS_PER_W = 80
CHUNK_IT = 40
AGG_ROWS = 10240
DUMMY_ROW = 10000


def _dot(a, b):
    return lax.dot_general(a, b, (((a.ndim - 1,), (0,)), ((), ())),
                           preferred_element_type=F32)


_dot_fast = _dot


def _bn(x, g, b):
    mu = jnp.mean(x, axis=0, keepdims=True)
    var = jnp.mean((x - mu) ** 2, axis=0, keepdims=True)
    return (x - mu) / jnp.sqrt(var + BN_EPS) * g + b


def _sc_msgpass_body(h_hbm, e_hbm, src_hbm, dst_hbm, out_hbm,
                     aggr_sh, src_v, dst_v, hbuf, ebuf, sem):
    c = lax.axis_index("c")
    s = lax.axis_index("s")
    wid = c * NS + s

    def zrow(i, _):
        for j in range(8):
            hbuf[i, pl.ds(j * 16, 16)] = jnp.zeros((16,), F32)
        return 0
    lax.fori_loop(0, 128, zrow, 0)
    base_r = s * (AGG_ROWS // NS)
    for k in range(5):
        pltpu.sync_copy(hbuf, aggr_sh.at[pl.ds(base_r + k * 128, 128)])
    plsc.subcore_barrier()

    def chunk(it, _):
        r0 = wid * ROWS_PER_W + it
        pltpu.sync_copy(src_hbm.at[pl.ds(r0, 1)], src_v)
        pltpu.sync_copy(dst_hbm.at[pl.ds(r0, 1)], dst_v)
        d0 = pltpu.async_copy(h_hbm.at[src_v.at[0]], hbuf, sem)
        pltpu.sync_copy(e_hbm.at[pl.ds(r0 * 128, 128)], ebuf)
        d0.wait()

        def row(i, _):
            for j in range(8):
                sl = pl.ds(j * 16, 16)
                hbuf[i, sl] = jnp.maximum(hbuf[i, sl] + ebuf[i, sl], 0.0)
            return 0
        lax.fori_loop(0, 128, row, 0)
        pltpu.sync_copy(hbuf, aggr_sh.at[dst_v.at[0]], add=True)
        return 0
    lax.fori_loop(0, ROWS_PER_W, chunk, 0)
    plsc.subcore_barrier()

    rows = AGG_ROWS // NS
    pltpu.sync_copy(aggr_sh.at[pl.ds(s * rows, rows)],
                    out_hbm.at[pl.ds(c * AGG_ROWS + s * rows, rows)])


@functools.cache
def _get_sc_msgpass():
    return pl.kernel(
        _sc_msgpass_body,
        out_type=jax.ShapeDtypeStruct((2 * AGG_ROWS, HIDDEN), F32),
        mesh=plsc.VectorSubcoreMesh(core_axis_name="c", subcore_axis_name="s",
                                    num_cores=NC, num_subcores=NS),
        scratch_types=[
            pltpu.VMEM_SHARED((AGG_ROWS, HIDDEN), F32),
            pltpu.VMEM((1, 128), jnp.int32),
            pltpu.VMEM((1, 128), jnp.int32),
            pltpu.VMEM((128, HIDDEN), F32),
            pltpu.VMEM((128, HIDDEN), F32),
            pltpu.SemaphoreType.DMA,
        ],
    )


def _proj_body(x_ref, w_ref, b_ref, o_ref):
    o_ref[...] = _dot(x_ref[...], w_ref[...]) + b_ref[...]


def _edge_body(ea_ref, w0, w1, w2, b0, b1, b2, o0, o1, o2):
    ea = ea_ref[...]
    o0[...] = _dot_fast(ea, w0[...]) + b0[...]
    o1[...] = _dot_fast(ea, w1[...]) + b1[...]
    o2[...] = _dot_fast(ea, w2[...]) + b2[...]


def _layer_body(h_ref, a_ref, w1, b1, g1, bb1, w2, b2, g2, bb2, o_ref):
    a = a_ref[...]
    z = h_ref[...] + a[:N_NODES] + a[AGG_ROWS:AGG_ROWS + N_NODES]
    y = _dot(z, w1[...]) + b1[...]
    y = jnp.maximum(_bn(y, g1[...], bb1[...]), 0.0)
    y = _dot(y, w2[...]) + b2[...]
    o_ref[...] = jnp.maximum(_bn(y, g2[...], bb2[...]), 0.0)


def _dot0(a, b):
    return lax.dot_general(a, b, (((0,), (0,)), ((), ())),
                           precision=lax.Precision.HIGHEST,
                           preferred_element_type=F32)


HEAD_BLK = 2000
HEAD_STEPS = N_NODES // HEAD_BLK


def _head_body(h_ref, bc_ref, gum_ref, hw1, hb1, hw2, hb2, bmw, bmb,
               lng, lnb, w1a, w1b, cb1, cg, cb, cw2, cb2, o_ref,
               sg, scnt, snum, sden):
    i = pl.program_id(0)
    h = h_ref[...]
    hh = jnp.maximum(_dot(h, hw1[...]) + hb1[...], 0.0)
    a = _dot(hh, hw2[...]) + hb2[...] + gum_ref[...]
    m = jnp.max(a, axis=-1, keepdims=True)
    ex = jnp.exp(a - m)
    assign = ex / jnp.sum(ex, axis=-1, keepdims=True)

    bc = bc_ref[...]
    ids = lax.broadcasted_iota(jnp.int32, (HEAD_BLK, N_GRAPHS), 1)
    oh = (ids == bc).astype(F32)
    ones = jnp.ones((HEAD_BLK, 1), F32)
    a2 = jnp.concatenate(
        [assign[:, cc:cc + 1] * oh for cc in range(N_BLOBS)], axis=1)

    g_c = _dot0(oh, h)
    cnt_c = _dot0(oh, ones)
    num_c = _dot0(a2, h)
    den_c = _dot0(a2, ones)

    @pl.when(i == 0)
    def _init():
        sg[...] = g_c
        scnt[...] = cnt_c
        snum[...] = num_c
        sden[...] = den_c

    @pl.when(i > 0)
    def _acc():
        sg[...] += g_c
        scnt[...] += cnt_c
        snum[...] += num_c
        sden[...] += den_c

    @pl.when(i == HEAD_STEPS - 1)
    def _final():
        gemb = sg[...] / jnp.maximum(scnt[...], 1.0)
        blobs = snum[...] / (sden[...] + 1e-8)
        y = jnp.maximum(_dot(blobs, bmw[...]) + bmb[...], 0.0)
        mu = jnp.mean(y, axis=-1, keepdims=True)
        var = jnp.mean((y - mu) ** 2, axis=-1, keepdims=True)
        y = (y - mu) / jnp.sqrt(var + LN_EPS) * lng[...] + lnb[...]
        bm = y[0:N_GRAPHS]
        for cc in range(1, N_BLOBS):
            bm = jnp.maximum(bm, y[cc * N_GRAPHS:(cc + 1) * N_GRAPHS])
        cz = _dot(gemb, w1a[...]) + _dot(bm, w1b[...]) + cb1[...]
        cz = jnp.maximum(_bn(cz, cg[...], cb[...]), 0.0)
        o_ref[...] = _dot(cz, cw2[...]) + cb2[...]


def _row(v):
    return v.reshape(1, -1)


def kernel(x, edge_index, edge_attr, batch, params):
    p = params
    src = edge_index[0].astype(jnp.int32)
    dst = edge_index[1].astype(jnp.int32)
    srcp = jnp.concatenate([src, jnp.zeros((PAD_N,), jnp.int32)]).reshape(-1, 128)
    dstp = jnp.concatenate(
        [dst, jnp.full((PAD_N,), DUMMY_ROW, jnp.int32)]).reshape(-1, 128)
    eap = jnp.concatenate(
        [edge_attr, jnp.zeros((PAD_N, EDGE_DIM), F32)], axis=0)
    gum = jax.random.gumbel(jax.random.key(1234), (N_NODES, N_BLOBS), dtype=F32)
    bc = batch.astype(jnp.int32).reshape(N_NODES, 1)

    h = pl.pallas_call(
        _proj_body,
        out_shape=jax.ShapeDtypeStruct((N_NODES, HIDDEN), F32),
    )(x, p['proj_W'], _row(p['proj_b']))

    eb = 4096
    grid = E_PAD // eb
    full = lambda shape: pl.BlockSpec(shape, lambda i: (0, 0))
    e_specs = pl.BlockSpec((eb, EDGE_DIM), lambda i: (i, 0))
    o_specs = pl.BlockSpec((eb, HIDDEN), lambda i: (i, 0))
    lp = p['layers']
    e0, e1, e2 = pl.pallas_call(
        _edge_body,
        grid=(grid,),
        in_specs=[e_specs] + [full((EDGE_DIM, HIDDEN))] * 3 + [full((1, HIDDEN))] * 3,
        out_specs=[o_specs] * 3,
        out_shape=[jax.ShapeDtypeStruct((E_PAD, HIDDEN), F32)] * 3,
    )(eap, lp[0]['edge_W'], lp[1]['edge_W'], lp[2]['edge_W'],
      _row(lp[0]['edge_b']), _row(lp[1]['edge_b']), _row(lp[2]['edge_b']))

    layer_call = pl.pallas_call(
        _layer_body,
        out_shape=jax.ShapeDtypeStruct((N_NODES, HIDDEN), F32),
    )
    for li, e in enumerate((e0, e1, e2)):
        w = lp[li]
        aggr = _get_sc_msgpass()(h, e, srcp, dstp)
        h = layer_call(h, aggr, w['mlp_W1'], _row(w['mlp_b1']),
                       _row(w['mlp_bn_g']), _row(w['mlp_bn_b']),
                       w['mlp_W2'], _row(w['mlp_b2']),
                       _row(w['bn_g']), _row(w['bn_b']))

    fullh = lambda shape: pl.BlockSpec(shape, lambda i: tuple(0 for _ in shape))
    out = pl.pallas_call(
        _head_body,
        grid=(HEAD_STEPS,),
        in_specs=[pl.BlockSpec((HEAD_BLK, HIDDEN), lambda i: (i, 0)),
                  pl.BlockSpec((HEAD_BLK, 1), lambda i: (i, 0)),
                  pl.BlockSpec((HEAD_BLK, N_BLOBS), lambda i: (i, 0)),
                  fullh((HIDDEN, 64)), fullh((1, 64)),
                  fullh((64, N_BLOBS)), fullh((1, N_BLOBS)),
                  fullh((HIDDEN, HIDDEN)), fullh((1, HIDDEN)),
                  fullh((1, HIDDEN)), fullh((1, HIDDEN)),
                  fullh((HIDDEN, HIDDEN)), fullh((HIDDEN, HIDDEN)),
                  fullh((1, HIDDEN)), fullh((1, HIDDEN)), fullh((1, HIDDEN)),
                  fullh((HIDDEN, 10)), fullh((1, 10))],
        out_specs=pl.BlockSpec((N_GRAPHS, 10), lambda i: (0, 0)),
        scratch_shapes=[pltpu.VMEM((N_GRAPHS, HIDDEN), F32),
                        pltpu.VMEM((N_GRAPHS, 1), F32),
                        pltpu.VMEM((HIDDEN, HIDDEN), F32),
                        pltpu.VMEM((HIDDEN, 1), F32)],
        out_shape=jax.ShapeDtypeStruct((N_GRAPHS, 10), F32),
    )(h, bc, gum, p['head_W1'], _row(p['head_b1']),
      p['head_W2'], _row(p['head_b2']),
      p['bm_W'], _row(p['bm_b']), _row(p['ln_g']), _row(p['ln_b']),
      p['clf_W1'][:HIDDEN], p['clf_W1'][HIDDEN:], _row(p['clf_b1']),
      _row(p['clf_bn_g']), _row(p['clf_bn_b']),
      p['clf_W2'], _row(p['clf_b2']))
    return out
